# Initial kernel scaffold; baseline (speedup 1.0000x reference)
import jax
import jax.numpy as jnp
from jax.experimental import pallas as pl


def kernel(x, token_table, pos_table):
    raise NotImplementedError("write your pallas kernel here")


# speedup vs baseline: 5.2372x; 5.2372x over previous
import functools

import jax
import jax.numpy as jnp
from jax import lax
from jax.experimental import pallas as pl
from jax.experimental.pallas import tpu as pltpu
from jax.experimental.pallas import tpu_sc as plsc

VOCAB, MAXLEN, EMBED, BATCH = 100000, 200, 32, 4096
NC, NS = 2, 16
NW = NC * NS
ROWS_PER_W = BATCH // NW
CB = 4
CH = CB * MAXLEN
N_CHUNKS = ROWS_PER_W // CB
PER_W = ROWS_PER_W * MAXLEN


def _build():
  mesh = plsc.VectorSubcoreMesh(core_axis_name="c", subcore_axis_name="s")

  @functools.partial(
      pl.kernel,
      out_type=jax.ShapeDtypeStruct((BATCH * MAXLEN, EMBED), jnp.float32),
      mesh=mesh,
      compiler_params=pltpu.CompilerParams(use_tc_tiling_on_sc=False),
      scratch_types=[
          pltpu.VMEM((PER_W,), jnp.int32),
          pltpu.VMEM((MAXLEN, EMBED), jnp.float32),
          pltpu.VMEM((2, CH, EMBED), jnp.float32),
          pltpu.SemaphoreType.DMA,
          pltpu.SemaphoreType.DMA,
          pltpu.SemaphoreType.DMA,
          pltpu.SemaphoreType.DMA,
      ],
  )
  def emb_kernel(tok_hbm, xf_hbm, pos_hbm, out_hbm,
                 idx_v, pos_v, rows_v, g0, g1, s0, s1):
    gsem = (g0, g1)
    ssem = (s0, s1)
    wid = lax.axis_index("s") * NC + lax.axis_index("c")
    base = wid * PER_W

    pltpu.sync_copy(xf_hbm.at[pl.ds(base, PER_W)], idx_v)
    pltpu.sync_copy(pos_hbm, pos_v)

    def gather_desc(k, b):
      off = pl.multiple_of(k * CH, CH)
      return pltpu.make_async_copy(
          tok_hbm.at[idx_v.at[pl.ds(off, CH)]], rows_v.at[b], gsem[b])

    def store_desc(k, b):
      off = pl.multiple_of(base + k * CH, CH)
      return pltpu.make_async_copy(
          rows_v.at[b], out_hbm.at[pl.ds(off, CH)], ssem[b])

    gather_desc(0, 0).start()

    @pl.loop(0, N_CHUNKS, step=2)
    def _chunks(k0):
      for b in range(2):
        k = k0 + b
        nb = 1 - b

        @pl.when(k + 1 < N_CHUNKS)
        def _issue_next():
          @pl.when(k >= 1)
          def _drain_prev_store():
            store_desc(k - 1, nb).wait()
          gather_desc(k + 1, nb).start()

        gather_desc(k, b).wait()

        rows_b = rows_v.at[b]

        @pl.loop(0, MAXLEN)
        def _add(t):
          p0 = pos_v[t, pl.ds(0, 16)]
          p1 = pos_v[t, pl.ds(16, 16)]
          for c in range(CB):
            r = c * MAXLEN + t
            rows_b[r, pl.ds(0, 16)] = rows_b[r, pl.ds(0, 16)] + p0
            rows_b[r, pl.ds(16, 16)] = rows_b[r, pl.ds(16, 16)] + p1

        store_desc(k, b).start()

    store_desc(N_CHUNKS - 2, 0).wait()
    store_desc(N_CHUNKS - 1, 1).wait()

  return emb_kernel


_emb = _build()


def kernel(x, token_table, pos_table):
  xf = x.reshape(-1).astype(jnp.int32)
  out = _emb(token_table, xf, pos_table)
  return out.reshape(BATCH, MAXLEN, EMBED)


# speedup vs baseline: 5.2402x; 1.0006x over previous
import functools

import jax
import jax.numpy as jnp
from jax import lax
from jax.experimental import pallas as pl
from jax.experimental.pallas import tpu as pltpu
from jax.experimental.pallas import tpu_sc as plsc

VOCAB, MAXLEN, EMBED, BATCH = 100000, 200, 32, 4096
NC, NS = 2, 16
NW = NC * NS
ROWS_PER_W = BATCH // NW
CB = 4
CH = CB * MAXLEN
N_CHUNKS = ROWS_PER_W // CB
PER_W = ROWS_PER_W * MAXLEN
ROW128 = CH * EMBED // 128
TPG = MAXLEN // 4


def _build():
  mesh = plsc.VectorSubcoreMesh(core_axis_name="c", subcore_axis_name="s")

  @functools.partial(
      pl.kernel,
      out_type=jax.ShapeDtypeStruct((BATCH * MAXLEN * EMBED // 128, 128),
                                    jnp.float32),
      mesh=mesh,
      compiler_params=pltpu.CompilerParams(use_tc_tiling_on_sc=False),
      scratch_types=[
          pltpu.VMEM((2, CH), jnp.int32),
          pltpu.VMEM((MAXLEN, EMBED), jnp.float32),
          pltpu.VMEM((2, CH, EMBED), jnp.float32),
          pltpu.VMEM((2, ROW128, 128), jnp.float32),
          pltpu.SemaphoreType.DMA,
          pltpu.SemaphoreType.DMA,
          pltpu.SemaphoreType.DMA,
          pltpu.SemaphoreType.DMA,
          pltpu.SemaphoreType.DMA,
          pltpu.SemaphoreType.DMA,
      ],
  )
  def emb_kernel(tok_hbm, xf_hbm, pos_hbm, out_hbm,
                 idx_v, pos_v, rows_v, st_v, g0, g1, s0, s1, i0, i1):
    gsem = (g0, g1)
    ssem = (s0, s1)
    isem = (i0, i1)
    wid = lax.axis_index("s") * NC + lax.axis_index("c")
    base = wid * PER_W

    pltpu.sync_copy(pos_hbm, pos_v)

    def idx_desc(k, b):
      off = pl.multiple_of(base + k * CH, CH)
      return pltpu.make_async_copy(
          xf_hbm.at[pl.ds(off, CH)], idx_v.at[b], isem[b])

    def gather_desc(b):
      return pltpu.make_async_copy(
          tok_hbm.at[idx_v.at[b]], rows_v.at[b], gsem[b])

    def store_desc(k, b):
      off = pl.multiple_of((base + k * CH) * EMBED // 128, ROW128)
      return pltpu.make_async_copy(
          st_v.at[b], out_hbm.at[pl.ds(off, ROW128)], ssem[b])

    idx_desc(0, 0).start()
    idx_desc(0, 0).wait()
    gather_desc(0).start()
    idx_desc(1, 1).start()

    @pl.loop(0, N_CHUNKS, step=2)
    def _chunks(k0):
      for b in range(2):
        k = k0 + b
        nb = 1 - b

        gather_desc(b).wait()

        @pl.when(k + 2 < N_CHUNKS)
        def _prefetch_idx():
          idx_desc(k + 2, b).start()

        @pl.when(k + 1 < N_CHUNKS)
        def _issue_next_gather():
          idx_desc(k + 1, nb).wait()
          gather_desc(nb).start()

        @pl.when(k >= 2)
        def _drain_store():
          store_desc(k - 2, b).wait()

        rows_b = rows_v.at[b]
        st_b = st_v.at[b]

        @pl.loop(0, MAXLEN, step=4)
        def _add(t0):
          s = t0 // 4
          for dt in range(4):
            t = t0 + dt
            p0 = pos_v[t, pl.ds(0, 16)]
            p1 = pos_v[t, pl.ds(16, 16)]
            for c in range(CB):
              r = c * MAXLEN + t
              row = c * TPG + s
              st_b[row, pl.ds(dt * 32, 16)] = rows_b[r, pl.ds(0, 16)] + p0
              st_b[row, pl.ds(dt * 32 + 16, 16)] = (
                  rows_b[r, pl.ds(16, 16)] + p1)

        store_desc(k, b).start()

    store_desc(N_CHUNKS - 2, 0).wait()
    store_desc(N_CHUNKS - 1, 1).wait()

  return emb_kernel


_emb = _build()


def kernel(x, token_table, pos_table):
  xf = x.reshape(-1).astype(jnp.int32)
  out = _emb(token_table, xf, pos_table)
  return out.reshape(BATCH, MAXLEN, EMBED)


# speedup vs baseline: 11.8030x; 2.2524x over previous
import functools

import jax
import jax.numpy as jnp
from jax import lax
from jax.experimental import pallas as pl
from jax.experimental.pallas import tpu as pltpu
from jax.experimental.pallas import tpu_sc as plsc

VOCAB, MAXLEN, EMBED, BATCH = 100000, 200, 32, 4096
NC, NS = 2, 16
NW = NC * NS
BW = BATCH // NW
TC = 5
N_CHUNKS = MAXLEN // TC
PAD = 133


def _build():
  mesh = plsc.VectorSubcoreMesh(core_axis_name="c", subcore_axis_name="s")

  @functools.partial(
      pl.kernel,
      out_type=jax.ShapeDtypeStruct((MAXLEN, EMBED // 8, BATCH // 128, 8, 128),
                                    jnp.float32),
      mesh=mesh,
      compiler_params=pltpu.CompilerParams(use_tc_tiling_on_sc=False,
                                           needs_layout_passes=False),
      scratch_types=[
          pltpu.VMEM((2, TC, BW), jnp.int32),
          pltpu.VMEM((MAXLEN, EMBED), jnp.float32),
          pltpu.VMEM((2, TC, BW, EMBED), jnp.float32),
          pltpu.VMEM((2, TC * 4 * 8, PAD), jnp.float32),
          pltpu.SemaphoreType.DMA,
          pltpu.SemaphoreType.DMA,
          pltpu.SemaphoreType.DMA,
          pltpu.SemaphoreType.DMA,
          pltpu.SemaphoreType.DMA,
          pltpu.SemaphoreType.DMA,
      ],
  )
  def emb_kernel(tok_hbm, xt_hbm, pos_hbm, out_hbm,
                 idx_v, pos_v, rows_v, st_v, g0, g1, s0, s1, i0, i1):
    gsem = (g0, g1)
    ssem = (s0, s1)
    isem = (i0, i1)
    wid = lax.axis_index("s") * NC + lax.axis_index("c")
    bbase = wid * BW

    pltpu.sync_copy(pos_hbm, pos_v)

    lane = lax.iota(jnp.int32, 16)
    i_eg0 = lane // 8
    i_eg1 = i_eg0 + 2
    i_el = lane % 8

    def idx_desc(k, b):
      return pltpu.make_async_copy(
          xt_hbm.at[pl.ds(k * TC, TC), pl.ds(bbase, BW)],
          idx_v.at[b], isem[b])

    def gather_descs(b):
      return [pltpu.make_async_copy(
                  tok_hbm.at[idx_v.at[b, ti]], rows_v.at[b, ti], gsem[b])
              for ti in range(TC)]

    def store_descs(k, b):
      return [pltpu.make_async_copy(
                  st_v.at[b, pl.ds((ti * 4 + eg) * 8, 8), pl.ds(0, 128)],
                  out_hbm.at[k * TC + ti, eg, wid], ssem[b])
              for ti in range(TC) for eg in range(4)]

    idx_desc(0, 0).start()
    idx_desc(0, 0).wait()
    for d in gather_descs(0):
      d.start()
    idx_desc(1, 1).start()

    @pl.loop(0, N_CHUNKS, step=2)
    def _chunks(k0):
      for b in range(2):
        k = k0 + b
        nb = 1 - b

        for d in gather_descs(b):
          d.wait()

        @pl.when(k + 2 < N_CHUNKS)
        def _prefetch_idx():
          idx_desc(k + 2, b).start()

        @pl.when(k + 1 < N_CHUNKS)
        def _issue_next_gather():
          idx_desc(k + 1, nb).wait()
          for d in gather_descs(nb):
            d.start()

        @pl.when(k >= 2)
        def _drain_store():
          for d in store_descs(k - 2, b):
            d.wait()

        st_b = st_v.at[b]

        for ti in range(TC):
          t = k * TC + ti
          p0 = pos_v[t, pl.ds(0, 16)]
          p1 = pos_v[t, pl.ds(16, 16)]
          i_row0 = ti * 32 + lane
          i_row1 = i_row0 + 16

          @pl.loop(0, BW, step=8)
          def _rows(bl0):
            for d in range(8):
              bl = bl0 + d
              i_bl = jnp.full((16,), 0, jnp.int32) + bl
              r0 = rows_v[b, ti, bl, pl.ds(0, 16)]
              r1 = rows_v[b, ti, bl, pl.ds(16, 16)]
              plsc.store_scatter(st_b, [i_row0, i_bl], r0 + p0)
              plsc.store_scatter(st_b, [i_row1, i_bl], r1 + p1)

        for d in store_descs(k, b):
          d.start()

    for d in store_descs(N_CHUNKS - 2, 0):
      d.wait()
    for d in store_descs(N_CHUNKS - 1, 1):
      d.wait()

  return emb_kernel


_emb = _build()


def kernel(x, token_table, pos_table):
  xt = x.T.astype(jnp.int32)
  out5 = _emb(token_table, xt, pos_table)
  return out5.transpose(2, 4, 0, 1, 3).reshape(BATCH, MAXLEN, EMBED)
